# baseline (speedup 1.0000x reference)
import functools

import jax
import jax.numpy as jnp
from jax import lax
from jax.experimental import pallas as pl
from jax.experimental.pallas import tpu as pltpu
from jax.experimental.pallas import tpu_sc as plsc

NPTS = 8192
NNB = 16
CF = 256
NROWS = NPTS * NNB
BP = 128
BR = BP * NNB
HI = lax.Precision.HIGHEST
F32 = jnp.float32
BIGF = 3.0e38
BIGI = 2**30


def _mm(a, b):
    return jax.lax.dot_general(a, b, (((1,), (0,)), ((), ())),
                               preferred_element_type=F32, precision=HI)


def _mmb(a, b):
    return jax.lax.dot_general(a.astype(jnp.bfloat16), b.astype(jnp.bfloat16),
                               (((1,), (0,)), ((), ())),
                               preferred_element_type=F32)


def _xq_body(x_ref, wt_ref, b_ref, o_ref):
    o_ref[...] = _mmb(x_ref[...], wt_ref[...]) + b_ref[...]


def _run_xq(x, WqT, bq_row):
    return pl.pallas_call(
        _xq_body,
        grid=(16,),
        in_specs=[
            pl.BlockSpec((512, CF), lambda i: (i, 0)),
            pl.BlockSpec((CF, CF), lambda i: (0, 0)),
            pl.BlockSpec((1, CF), lambda i: (0, 0)),
        ],
        out_specs=pl.BlockSpec((512, CF), lambda i: (i, 0)),
        out_shape=jax.ShapeDtypeStruct((NPTS, CF), F32),
    )(x, WqT, bq_row)


def _knn_body(pb_ref, pbT_ref, pf_ref, pfT_ref, o_ref, d2_ref):
    dot = jax.lax.dot_general(pb_ref[...], pbT_ref[...],
                              (((1,), (0,)), ((), ())),
                              preferred_element_type=F32)
    pf = pf_ref[...]
    px, py, pz = pf[:, 0:1], pf[:, 1:2], pf[:, 2:3]
    sqc = (px * px + py * py) + pz * pz
    pfT = pfT_ref[...]
    qx, qy, qz = pfT[0:1, :], pfT[1:2, :], pfT[2:3, :]
    sqr = (qx * qx + qy * qy) + qz * qz
    d2_ref[...] = (sqc + sqr) - 2.0 * dot
    rows = jax.lax.broadcasted_iota(jnp.int32, (NPTS, 128), 0)

    def rnd(k, carry):
        tv, ti = carry
        d2 = d2_ref[...]
        elig = (d2 > tv) | ((d2 == tv) & (rows > ti))
        v = jnp.where(elig, d2, BIGF)
        r = rows
        h = NPTS // 2
        while h >= 1:
            a, b = v[:h], v[h:]
            ra, rb = r[:h], r[h:]
            le = (a < b) | ((a == b) & (ra < rb))
            v = jnp.where(le, a, b)
            r = jnp.where(le, ra, rb)
            h //= 2
        o_ref[pl.ds(k, 1), :] = r
        return (v, r)

    lax.fori_loop(0, NNB, rnd,
                  (jnp.full((1, 128), -BIGF, F32),
                   jnp.full((1, 128), -1, jnp.int32)))


def _run_knn(p16b, p16bT, p16, p16T):
    return pl.pallas_call(
        _knn_body,
        grid=(64,),
        in_specs=[
            pl.BlockSpec((NPTS, 16), lambda i: (0, 0)),
            pl.BlockSpec((16, 128), lambda i: (0, i)),
            pl.BlockSpec((NPTS, 16), lambda i: (0, 0)),
            pl.BlockSpec((16, 128), lambda i: (0, i)),
        ],
        out_specs=pl.BlockSpec((16, 128), lambda i: (0, i)),
        out_shape=jax.ShapeDtypeStruct((NNB, NPTS), jnp.int32),
        scratch_shapes=[pltpu.VMEM((NPTS, 128), F32)],
    )(p16b, p16bT, p16, p16T)


CW = 384


def _make_gather():
    info = plsc.get_sparse_core_info()
    ncores, nsub = info.num_cores, info.num_subcores
    nworkers = ncores * nsub
    rows_per = NROWS // nworkers
    chunk = 128
    nchunks = rows_per // chunk
    mesh = plsc.VectorSubcoreMesh(core_axis_name="c", subcore_axis_name="s")

    @functools.partial(
        pl.kernel, mesh=mesh,
        out_type=jax.ShapeDtypeStruct((NROWS, CW), F32),
        scratch_types=[
            pltpu.VMEM((chunk,), jnp.int32),
            pltpu.VMEM((chunk, CW), F32),
            pltpu.SemaphoreType.DMA,
        ],
    )
    def gather(xp_hbm, idx_hbm, gxp_hbm, idx_v, rows_v, sem1):
        wid = lax.axis_index("s") * ncores + lax.axis_index("c")

        def body(c, carry):
            base = wid * rows_per + c * chunk
            pltpu.sync_copy(idx_hbm.at[pl.ds(base, chunk)], idx_v)
            pltpu.async_copy(xp_hbm.at[idx_v], rows_v, sem1).wait()
            pltpu.sync_copy(rows_v, gxp_hbm.at[pl.ds(base, chunk)])
            return carry

        lax.fori_loop(0, nchunks, body, 0)

    return gather


def _pr_block(gp_ref, pq_ref, w1t_ref, a1_ref, c1_ref, w2t_ref, b2_ref):
    gp3 = gp_ref[...][:, :16].reshape(BP, NNB, 16) - pq_ref[...][:, None, :]
    y1 = _mm(gp3.reshape(BR, 16), w1t_ref[...])
    z1 = jnp.maximum(y1 * a1_ref[...] + c1_ref[...], 0.0)
    return _mmb(z1, w2t_ref[...]) + b2_ref[...]


def _acc_stats(s_ref, vals, width):
    s = jnp.sum(vals, axis=0, keepdims=True)
    q = jnp.sum(vals * vals, axis=0, keepdims=True)
    part = jnp.concatenate([s, q, jnp.zeros((6, width), F32)], axis=0)

    @pl.when(pl.program_id(0) == 0)
    def _():
        s_ref[...] = jnp.zeros_like(s_ref)

    s_ref[...] += part


def _stats1_body(gp_ref, pq_ref, w1t_ref, s_ref):
    gp3 = gp_ref[...][:, :16].reshape(BP, NNB, 16) - pq_ref[...][:, None, :]
    y1 = _mm(gp3.reshape(BR, 16), w1t_ref[...])
    _acc_stats(s_ref, y1, 16)


def _run_stats1(g_p, p16, W1T):
    return pl.pallas_call(
        _stats1_body,
        grid=(NPTS // BP,),
        in_specs=[
            pl.BlockSpec((BR, 128), lambda i: (i, 2)),
            pl.BlockSpec((BP, 16), lambda i: (i, 0)),
            pl.BlockSpec((16, 16), lambda i: (0, 0)),
        ],
        out_specs=pl.BlockSpec((8, 16), lambda i: (0, 0)),
        out_shape=jax.ShapeDtypeStruct((8, 16), F32),
    )(g_p, p16, W1T)


def _w0_block(gx_ref, gp_ref, xq_ref, pq_ref, wkt_ref, bk_ref,
              w1t_ref, a1_ref, c1_ref, w2t_ref, b2_ref):
    g_k = _mmb(gx_ref[...], wkt_ref[...]) + bk_ref[...]
    p_r = _pr_block(gp_ref, pq_ref, w1t_ref, a1_ref, c1_ref, w2t_ref, b2_ref)
    w03 = (g_k.reshape(BP, NNB, CF) - xq_ref[...][:, None, :]
           + p_r.reshape(BP, NNB, CF))
    return w03.reshape(BR, CF), p_r


def _alpha_body(gx_ref, gp_ref, xq_ref, pq_ref, wkt_ref, bk_ref,
                w1t_ref, a1_ref, c1_ref, w2t_ref, b2_ref, s_ref):
    w0, _ = _w0_block(gx_ref, gp_ref, xq_ref, pq_ref, wkt_ref, bk_ref,
                      w1t_ref, a1_ref, c1_ref, w2t_ref, b2_ref)
    _acc_stats(s_ref, w0, CF)


def _big_specs():
    return [
        pl.BlockSpec((BR, CF), lambda i: (i, 0)),
        pl.BlockSpec((BR, 128), lambda i: (i, 2)),
        pl.BlockSpec((BP, CF), lambda i: (i, 0)),
        pl.BlockSpec((BP, 16), lambda i: (i, 0)),
        pl.BlockSpec((CF, CF), lambda i: (0, 0)),
        pl.BlockSpec((1, CF), lambda i: (0, 0)),
        pl.BlockSpec((16, 16), lambda i: (0, 0)),
        pl.BlockSpec((1, 16), lambda i: (0, 0)),
        pl.BlockSpec((1, 16), lambda i: (0, 0)),
        pl.BlockSpec((16, CF), lambda i: (0, 0)),
        pl.BlockSpec((1, CF), lambda i: (0, 0)),
    ]


def _run_alpha(args):
    return pl.pallas_call(
        _alpha_body,
        grid=(NPTS // BP,),
        in_specs=_big_specs(),
        out_specs=pl.BlockSpec((8, CF), lambda i: (0, 0)),
        out_shape=jax.ShapeDtypeStruct((8, CF), F32),
    )(*args)


def _beta_body(gx_ref, gp_ref, xq_ref, pq_ref, wkt_ref, bk_ref,
               w1t_ref, a1_ref, c1_ref, w2t_ref, b2_ref,
               a2_ref, c2_ref, wc1t_ref, bc1_ref, w1o_ref, s_ref):
    w0, _ = _w0_block(gx_ref, gp_ref, xq_ref, pq_ref, wkt_ref, bk_ref,
                      w1t_ref, a1_ref, c1_ref, w2t_ref, b2_ref)
    z2 = jnp.maximum(w0 * a2_ref[...] + c2_ref[...], 0.0)
    w1 = _mmb(z2, wc1t_ref[...]) + bc1_ref[...]
    w1o_ref[...] = w1
    _acc_stats(s_ref, w1, 32)


def _run_beta(args):
    specs = _big_specs() + [
        pl.BlockSpec((1, CF), lambda i: (0, 0)),
        pl.BlockSpec((1, CF), lambda i: (0, 0)),
        pl.BlockSpec((CF, 32), lambda i: (0, 0)),
        pl.BlockSpec((1, 32), lambda i: (0, 0)),
    ]
    return pl.pallas_call(
        _beta_body,
        grid=(NPTS // BP,),
        in_specs=specs,
        out_specs=[
            pl.BlockSpec((BR, 32), lambda i: (i, 0)),
            pl.BlockSpec((8, 32), lambda i: (0, 0)),
        ],
        out_shape=[
            jax.ShapeDtypeStruct((NROWS, 32), F32),
            jax.ShapeDtypeStruct((8, 32), F32),
        ],
    )(*args)


def _gamma_body(w1_ref, gx_ref, gp_ref, pq_ref, wvt_ref, bv_ref,
                w1t_ref, a1_ref, c1_ref, w2t_ref, b2_ref,
                a3_ref, c3_ref, wc2t_ref, bc2_ref, o_ref):
    z3 = jnp.maximum(w1_ref[...] * a3_ref[...] + c3_ref[...], 0.0)
    w2 = _mm(z3, wc2t_ref[...]) + bc2_ref[...]
    w23 = w2.reshape(BP, NNB, 32)
    mx = jnp.max(w23, axis=1, keepdims=True)
    ex = jnp.exp(w23 - mx)
    att = ex / jnp.sum(ex, axis=1, keepdims=True)
    att_t = jnp.concatenate([att] * 8, axis=2)
    g_v = _mmb(gx_ref[...], wvt_ref[...]) + bv_ref[...]
    p_r = _pr_block(gp_ref, pq_ref, w1t_ref, a1_ref, c1_ref, w2t_ref, b2_ref)
    h = (g_v + p_r).reshape(BP, NNB, CF)
    o_ref[...] = jnp.sum(h * att_t, axis=1)


def _run_gamma(w1, gx, gp, p16, WvT, bv_row, W1T, A1, C1, W2T, bp2_row,
               A3, C3, Wc2T, bc2_row):
    specs = [
        pl.BlockSpec((BR, 32), lambda i: (i, 0)),
        pl.BlockSpec((BR, CF), lambda i: (i, 0)),
        pl.BlockSpec((BR, 128), lambda i: (i, 2)),
        pl.BlockSpec((BP, 16), lambda i: (i, 0)),
        pl.BlockSpec((CF, CF), lambda i: (0, 0)),
        pl.BlockSpec((1, CF), lambda i: (0, 0)),
        pl.BlockSpec((16, 16), lambda i: (0, 0)),
        pl.BlockSpec((1, 16), lambda i: (0, 0)),
        pl.BlockSpec((1, 16), lambda i: (0, 0)),
        pl.BlockSpec((16, CF), lambda i: (0, 0)),
        pl.BlockSpec((1, CF), lambda i: (0, 0)),
        pl.BlockSpec((1, 32), lambda i: (0, 0)),
        pl.BlockSpec((1, 32), lambda i: (0, 0)),
        pl.BlockSpec((32, 32), lambda i: (0, 0)),
        pl.BlockSpec((1, 32), lambda i: (0, 0)),
    ]
    return pl.pallas_call(
        _gamma_body,
        grid=(NPTS // BP,),
        in_specs=specs,
        out_specs=pl.BlockSpec((BP, CF), lambda i: (i, 0)),
        out_shape=jax.ShapeDtypeStruct((NPTS, CF), F32),
    )(w1, gx, gp, p16, WvT, bv_row, W1T, A1, C1, W2T, bp2_row,
      A3, C3, Wc2T, bc2_row)


def kernel(p, x, o, Wq, bq, Wk, bk, Wv, bv, Wp1, bp1, g1, be1, Wp2, bp2,
           g2, be2, Wc1, bc1, g3, be3, Wc2, bc2):
    row = lambda v: v[None, :]
    p16 = jnp.pad(p, ((0, 0), (0, 13)))
    p16b = p16.astype(jnp.bfloat16)
    W1T = jnp.pad(Wp1, ((0, 13), (0, 13))).T
    W2T = jnp.pad(Wp2, ((0, 0), (0, 13))).T

    x_q = _run_xq(x, Wq.T, row(bq))
    idxT = _run_knn(p16b, p16b.T, p16, p16.T)
    idxf = idxT.T.reshape(-1)

    xp = jnp.concatenate([x, p16, jnp.zeros((NPTS, CW - CF - 16), F32)],
                         axis=1)
    g_xp = _make_gather()(xp, idxf)

    M = float(NROWS)
    st1 = _run_stats1(g_xp, p16, W1T)
    m1 = st1[0] / M
    v1 = st1[1] / M - m1 * m1
    A1 = jnp.pad(g1, (0, 13)) / jnp.sqrt(v1 + 1e-5)
    C1 = jnp.pad(be1, (0, 13)) - m1 * A1

    big = (g_xp, g_xp, x_q, p16, Wk.T, row(bk), W1T, row(A1), row(C1),
           W2T, row(bp2))
    st2 = _run_alpha(big)
    m2 = st2[0] / M
    v2 = st2[1] / M - m2 * m2
    A2 = g2 / jnp.sqrt(v2 + 1e-5)
    C2 = be2 - m2 * A2

    w1, st3 = _run_beta(big + (row(A2), row(C2), Wc1.T, row(bc1)))
    m3 = st3[0, :32] / M
    v3 = st3[1, :32] / M - m3 * m3
    A3 = g3 / jnp.sqrt(v3 + 1e-5)
    C3 = be3 - m3 * A3

    out = _run_gamma(w1, g_xp, g_xp, p16, Wv.T, row(bv), W1T, row(A1), row(C1),
                     W2T, row(bp2), row(A3), row(C3), Wc2.T, row(bc2))
    return out


# speedup vs baseline: 1.8369x; 1.8369x over previous
import functools

import jax
import jax.numpy as jnp
from jax import lax
from jax.experimental import pallas as pl
from jax.experimental.pallas import tpu as pltpu
from jax.experimental.pallas import tpu_sc as plsc

NPTS = 8192
NNB = 16
CF = 256
NROWS = NPTS * NNB
BP = 128
BR = BP * NNB
HI = lax.Precision.HIGHEST
F32 = jnp.float32
BIGF = 3.0e38
BIGI = 2**30


def _mm(a, b):
    return jax.lax.dot_general(a, b, (((1,), (0,)), ((), ())),
                               preferred_element_type=F32, precision=HI)


def _mmb(a, b):
    return jax.lax.dot_general(a.astype(jnp.bfloat16), b.astype(jnp.bfloat16),
                               (((1,), (0,)), ((), ())),
                               preferred_element_type=F32)


def _xq_body(x_ref, wt_ref, b_ref, o_ref):
    o_ref[...] = _mmb(x_ref[...], wt_ref[...]) + b_ref[...]


def _run_xq(x, WqT, bq_row):
    return pl.pallas_call(
        _xq_body,
        grid=(16,),
        in_specs=[
            pl.BlockSpec((512, CF), lambda i: (i, 0)),
            pl.BlockSpec((CF, CF), lambda i: (0, 0)),
            pl.BlockSpec((1, CF), lambda i: (0, 0)),
        ],
        out_specs=pl.BlockSpec((512, CF), lambda i: (i, 0)),
        out_shape=jax.ShapeDtypeStruct((NPTS, CF), F32),
    )(x, WqT, bq_row)


def _knn_body(pb_ref, pbT_ref, pf_ref, pfT_ref, o_ref, d2_ref):
    dot = jax.lax.dot_general(pb_ref[...], pbT_ref[...],
                              (((1,), (0,)), ((), ())),
                              preferred_element_type=F32)
    pf = pf_ref[...]
    px, py, pz = pf[:, 0:1], pf[:, 1:2], pf[:, 2:3]
    sqc = (px * px + py * py) + pz * pz
    pfT = pfT_ref[...]
    qx, qy, qz = pfT[0:1, :], pfT[1:2, :], pfT[2:3, :]
    sqr = (qx * qx + qy * qy) + qz * qz
    d2_ref[...] = (sqc + sqr) - 2.0 * dot
    rows = jax.lax.broadcasted_iota(jnp.int32, (NPTS, 128), 0)

    def rnd(k, carry):
        d2 = d2_ref[...]
        im = jnp.argmin(d2, axis=0).reshape(1, 128)
        o_ref[pl.ds(k, 1), :] = im
        d2_ref[...] = jnp.where(rows == im, BIGF, d2)
        return carry

    lax.fori_loop(0, NNB, rnd, 0)


def _run_knn(p16b, p16bT, p16, p16T):
    return pl.pallas_call(
        _knn_body,
        grid=(64,),
        in_specs=[
            pl.BlockSpec((NPTS, 16), lambda i: (0, 0)),
            pl.BlockSpec((16, 128), lambda i: (0, i)),
            pl.BlockSpec((NPTS, 16), lambda i: (0, 0)),
            pl.BlockSpec((16, 128), lambda i: (0, i)),
        ],
        out_specs=pl.BlockSpec((16, 128), lambda i: (0, i)),
        out_shape=jax.ShapeDtypeStruct((NNB, NPTS), jnp.int32),
        scratch_shapes=[pltpu.VMEM((NPTS, 128), F32)],
    )(p16b, p16bT, p16, p16T)


CW = 384


def _make_gather():
    info = plsc.get_sparse_core_info()
    ncores, nsub = info.num_cores, info.num_subcores
    nworkers = ncores * nsub
    rows_per = NROWS // nworkers
    chunk = 128
    nchunks = rows_per // chunk
    mesh = plsc.VectorSubcoreMesh(core_axis_name="c", subcore_axis_name="s")

    @functools.partial(
        pl.kernel, mesh=mesh,
        out_type=jax.ShapeDtypeStruct((NROWS, CW), F32),
        scratch_types=[
            pltpu.VMEM((chunk,), jnp.int32),
            pltpu.VMEM((chunk, CW), F32),
            pltpu.SemaphoreType.DMA,
        ],
    )
    def gather(xp_hbm, idx_hbm, gxp_hbm, idx_v, rows_v, sem1):
        wid = lax.axis_index("s") * ncores + lax.axis_index("c")

        def body(c, carry):
            base = wid * rows_per + c * chunk
            pltpu.sync_copy(idx_hbm.at[pl.ds(base, chunk)], idx_v)
            pltpu.async_copy(xp_hbm.at[idx_v], rows_v, sem1).wait()
            pltpu.sync_copy(rows_v, gxp_hbm.at[pl.ds(base, chunk)])
            return carry

        lax.fori_loop(0, nchunks, body, 0)

    return gather


def _pr_block(gp_ref, pq_ref, w1t_ref, a1_ref, c1_ref, w2t_ref, b2_ref):
    gp3 = gp_ref[...][:, :16].reshape(BP, NNB, 16) - pq_ref[...][:, None, :]
    y1 = _mm(gp3.reshape(BR, 16), w1t_ref[...])
    z1 = jnp.maximum(y1 * a1_ref[...] + c1_ref[...], 0.0)
    return _mmb(z1, w2t_ref[...]) + b2_ref[...]


def _acc_stats(s_ref, vals, width):
    s = jnp.sum(vals, axis=0, keepdims=True)
    q = jnp.sum(vals * vals, axis=0, keepdims=True)
    part = jnp.concatenate([s, q, jnp.zeros((6, width), F32)], axis=0)

    @pl.when(pl.program_id(0) == 0)
    def _():
        s_ref[...] = jnp.zeros_like(s_ref)

    s_ref[...] += part


def _stats1_body(gp_ref, pq_ref, w1t_ref, s_ref):
    gp3 = gp_ref[...][:, :16].reshape(BP, NNB, 16) - pq_ref[...][:, None, :]
    y1 = _mm(gp3.reshape(BR, 16), w1t_ref[...])
    _acc_stats(s_ref, y1, 16)


def _run_stats1(g_p, p16, W1T):
    return pl.pallas_call(
        _stats1_body,
        grid=(NPTS // BP,),
        in_specs=[
            pl.BlockSpec((BR, 128), lambda i: (i, 2)),
            pl.BlockSpec((BP, 16), lambda i: (i, 0)),
            pl.BlockSpec((16, 16), lambda i: (0, 0)),
        ],
        out_specs=pl.BlockSpec((8, 16), lambda i: (0, 0)),
        out_shape=jax.ShapeDtypeStruct((8, 16), F32),
    )(g_p, p16, W1T)


def _w0_block(gx_ref, gp_ref, xq_ref, pq_ref, wkt_ref, bk_ref,
              w1t_ref, a1_ref, c1_ref, w2t_ref, b2_ref):
    g_k = _mmb(gx_ref[...], wkt_ref[...]) + bk_ref[...]
    p_r = _pr_block(gp_ref, pq_ref, w1t_ref, a1_ref, c1_ref, w2t_ref, b2_ref)
    w03 = (g_k.reshape(BP, NNB, CF) - xq_ref[...][:, None, :]
           + p_r.reshape(BP, NNB, CF))
    return w03.reshape(BR, CF), p_r


def _alpha_body(gx_ref, gp_ref, xq_ref, pq_ref, wkt_ref, bk_ref,
                w1t_ref, a1_ref, c1_ref, w2t_ref, b2_ref, s_ref):
    w0, _ = _w0_block(gx_ref, gp_ref, xq_ref, pq_ref, wkt_ref, bk_ref,
                      w1t_ref, a1_ref, c1_ref, w2t_ref, b2_ref)
    _acc_stats(s_ref, w0, CF)


def _big_specs():
    return [
        pl.BlockSpec((BR, CF), lambda i: (i, 0)),
        pl.BlockSpec((BR, 128), lambda i: (i, 2)),
        pl.BlockSpec((BP, CF), lambda i: (i, 0)),
        pl.BlockSpec((BP, 16), lambda i: (i, 0)),
        pl.BlockSpec((CF, CF), lambda i: (0, 0)),
        pl.BlockSpec((1, CF), lambda i: (0, 0)),
        pl.BlockSpec((16, 16), lambda i: (0, 0)),
        pl.BlockSpec((1, 16), lambda i: (0, 0)),
        pl.BlockSpec((1, 16), lambda i: (0, 0)),
        pl.BlockSpec((16, CF), lambda i: (0, 0)),
        pl.BlockSpec((1, CF), lambda i: (0, 0)),
    ]


def _run_alpha(args):
    return pl.pallas_call(
        _alpha_body,
        grid=(NPTS // BP,),
        in_specs=_big_specs(),
        out_specs=pl.BlockSpec((8, CF), lambda i: (0, 0)),
        out_shape=jax.ShapeDtypeStruct((8, CF), F32),
    )(*args)


def _beta_body(gx_ref, gp_ref, xq_ref, pq_ref, wkt_ref, bk_ref,
               w1t_ref, a1_ref, c1_ref, w2t_ref, b2_ref,
               a2_ref, c2_ref, wc1t_ref, bc1_ref, w1o_ref, s_ref):
    w0, _ = _w0_block(gx_ref, gp_ref, xq_ref, pq_ref, wkt_ref, bk_ref,
                      w1t_ref, a1_ref, c1_ref, w2t_ref, b2_ref)
    z2 = jnp.maximum(w0 * a2_ref[...] + c2_ref[...], 0.0)
    w1 = _mmb(z2, wc1t_ref[...]) + bc1_ref[...]
    w1o_ref[...] = w1
    _acc_stats(s_ref, w1, 32)


def _run_beta(args):
    specs = _big_specs() + [
        pl.BlockSpec((1, CF), lambda i: (0, 0)),
        pl.BlockSpec((1, CF), lambda i: (0, 0)),
        pl.BlockSpec((CF, 32), lambda i: (0, 0)),
        pl.BlockSpec((1, 32), lambda i: (0, 0)),
    ]
    return pl.pallas_call(
        _beta_body,
        grid=(NPTS // BP,),
        in_specs=specs,
        out_specs=[
            pl.BlockSpec((BR, 32), lambda i: (i, 0)),
            pl.BlockSpec((8, 32), lambda i: (0, 0)),
        ],
        out_shape=[
            jax.ShapeDtypeStruct((NROWS, 32), F32),
            jax.ShapeDtypeStruct((8, 32), F32),
        ],
    )(*args)


def _gamma_body(w1_ref, gx_ref, gp_ref, pq_ref, wvt_ref, bv_ref,
                w1t_ref, a1_ref, c1_ref, w2t_ref, b2_ref,
                a3_ref, c3_ref, wc2t_ref, bc2_ref, o_ref):
    z3 = jnp.maximum(w1_ref[...] * a3_ref[...] + c3_ref[...], 0.0)
    w2 = _mm(z3, wc2t_ref[...]) + bc2_ref[...]
    w23 = w2.reshape(BP, NNB, 32)
    mx = jnp.max(w23, axis=1, keepdims=True)
    ex = jnp.exp(w23 - mx)
    att = ex / jnp.sum(ex, axis=1, keepdims=True)
    att_t = jnp.concatenate([att] * 8, axis=2)
    g_v = _mmb(gx_ref[...], wvt_ref[...]) + bv_ref[...]
    p_r = _pr_block(gp_ref, pq_ref, w1t_ref, a1_ref, c1_ref, w2t_ref, b2_ref)
    h = (g_v + p_r).reshape(BP, NNB, CF)
    o_ref[...] = jnp.sum(h * att_t, axis=1)


def _run_gamma(w1, gx, gp, p16, WvT, bv_row, W1T, A1, C1, W2T, bp2_row,
               A3, C3, Wc2T, bc2_row):
    specs = [
        pl.BlockSpec((BR, 32), lambda i: (i, 0)),
        pl.BlockSpec((BR, CF), lambda i: (i, 0)),
        pl.BlockSpec((BR, 128), lambda i: (i, 2)),
        pl.BlockSpec((BP, 16), lambda i: (i, 0)),
        pl.BlockSpec((CF, CF), lambda i: (0, 0)),
        pl.BlockSpec((1, CF), lambda i: (0, 0)),
        pl.BlockSpec((16, 16), lambda i: (0, 0)),
        pl.BlockSpec((1, 16), lambda i: (0, 0)),
        pl.BlockSpec((1, 16), lambda i: (0, 0)),
        pl.BlockSpec((16, CF), lambda i: (0, 0)),
        pl.BlockSpec((1, CF), lambda i: (0, 0)),
        pl.BlockSpec((1, 32), lambda i: (0, 0)),
        pl.BlockSpec((1, 32), lambda i: (0, 0)),
        pl.BlockSpec((32, 32), lambda i: (0, 0)),
        pl.BlockSpec((1, 32), lambda i: (0, 0)),
    ]
    return pl.pallas_call(
        _gamma_body,
        grid=(NPTS // BP,),
        in_specs=specs,
        out_specs=pl.BlockSpec((BP, CF), lambda i: (i, 0)),
        out_shape=jax.ShapeDtypeStruct((NPTS, CF), F32),
    )(w1, gx, gp, p16, WvT, bv_row, W1T, A1, C1, W2T, bp2_row,
      A3, C3, Wc2T, bc2_row)


def kernel(p, x, o, Wq, bq, Wk, bk, Wv, bv, Wp1, bp1, g1, be1, Wp2, bp2,
           g2, be2, Wc1, bc1, g3, be3, Wc2, bc2):
    row = lambda v: v[None, :]
    p16 = jnp.pad(p, ((0, 0), (0, 13)))
    p16b = p16.astype(jnp.bfloat16)
    W1T = jnp.pad(Wp1, ((0, 13), (0, 13))).T
    W2T = jnp.pad(Wp2, ((0, 0), (0, 13))).T

    x_q = _run_xq(x, Wq.T, row(bq))
    idxT = _run_knn(p16b, p16b.T, p16, p16.T)
    idxf = idxT.T.reshape(-1)

    xp = jnp.concatenate([x, p16, jnp.zeros((NPTS, CW - CF - 16), F32)],
                         axis=1)
    g_xp = _make_gather()(xp, idxf)

    M = float(NROWS)
    st1 = _run_stats1(g_xp, p16, W1T)
    m1 = st1[0] / M
    v1 = st1[1] / M - m1 * m1
    A1 = jnp.pad(g1, (0, 13)) / jnp.sqrt(v1 + 1e-5)
    C1 = jnp.pad(be1, (0, 13)) - m1 * A1

    big = (g_xp, g_xp, x_q, p16, Wk.T, row(bk), W1T, row(A1), row(C1),
           W2T, row(bp2))
    st2 = _run_alpha(big)
    m2 = st2[0] / M
    v2 = st2[1] / M - m2 * m2
    A2 = g2 / jnp.sqrt(v2 + 1e-5)
    C2 = be2 - m2 * A2

    w1, st3 = _run_beta(big + (row(A2), row(C2), Wc1.T, row(bc1)))
    m3 = st3[0, :32] / M
    v3 = st3[1, :32] / M - m3 * m3
    A3 = g3 / jnp.sqrt(v3 + 1e-5)
    C3 = be3 - m3 * A3

    out = _run_gamma(w1, g_xp, g_xp, p16, Wv.T, row(bv), W1T, row(A1), row(C1),
                     W2T, row(bp2), row(A3), row(C3), Wc2.T, row(bc2))
    return out


# speedup vs baseline: 2.2451x; 1.2222x over previous
import functools

import jax
import jax.numpy as jnp
from jax import lax
from jax.experimental import pallas as pl
from jax.experimental.pallas import tpu as pltpu
from jax.experimental.pallas import tpu_sc as plsc

NPTS = 8192
NNB = 16
CF = 256
NROWS = NPTS * NNB
BP = 128
BR = BP * NNB
HI = lax.Precision.HIGHEST
F32 = jnp.float32
BIGF = 3.0e38
BIGI = 2**30


def _mm(a, b):
    return jax.lax.dot_general(a, b, (((1,), (0,)), ((), ())),
                               preferred_element_type=F32, precision=HI)


def _mmb(a, b):
    return jax.lax.dot_general(a.astype(jnp.bfloat16), b.astype(jnp.bfloat16),
                               (((1,), (0,)), ((), ())),
                               preferred_element_type=F32)


def _xq_body(x_ref, wt_ref, b_ref, o_ref):
    o_ref[...] = _mmb(x_ref[...], wt_ref[...]) + b_ref[...]


def _run_xq(x, WqT, bq_row):
    return pl.pallas_call(
        _xq_body,
        grid=(16,),
        in_specs=[
            pl.BlockSpec((512, CF), lambda i: (i, 0)),
            pl.BlockSpec((CF, CF), lambda i: (0, 0)),
            pl.BlockSpec((1, CF), lambda i: (0, 0)),
        ],
        out_specs=pl.BlockSpec((512, CF), lambda i: (i, 0)),
        out_shape=jax.ShapeDtypeStruct((NPTS, CF), F32),
    )(x, WqT, bq_row)


def _knn_body(pb_ref, pbT_ref, pf_ref, pfT_ref, o_ref, d2_ref):
    dot = jax.lax.dot_general(pb_ref[...], pbT_ref[...],
                              (((1,), (0,)), ((), ())),
                              preferred_element_type=F32)
    pf = pf_ref[...]
    px, py, pz = pf[:, 0:1], pf[:, 1:2], pf[:, 2:3]
    sqc = (px * px + py * py) + pz * pz
    pfT = pfT_ref[...]
    qx, qy, qz = pfT[0:1, :], pfT[1:2, :], pfT[2:3, :]
    sqr = (qx * qx + qy * qy) + qz * qz
    d2_ref[...] = (sqc + sqr) - 2.0 * dot
    rows = jax.lax.broadcasted_iota(jnp.int32, (NPTS, 128), 0)

    def rnd(k, im_prev):
        d2m = jnp.where(rows == im_prev, BIGF, d2_ref[...])
        d2_ref[...] = d2m
        im = jnp.argmin(d2m, axis=0).reshape(1, 128)
        o_ref[pl.ds(k, 1), :] = im
        return im

    lax.fori_loop(0, NNB, rnd, jnp.full((1, 128), -1, jnp.int32))


def _run_knn(p16b, p16bT, p16, p16T):
    return pl.pallas_call(
        _knn_body,
        grid=(64,),
        in_specs=[
            pl.BlockSpec((NPTS, 16), lambda i: (0, 0)),
            pl.BlockSpec((16, 128), lambda i: (0, i)),
            pl.BlockSpec((NPTS, 16), lambda i: (0, 0)),
            pl.BlockSpec((16, 128), lambda i: (0, i)),
        ],
        out_specs=pl.BlockSpec((16, 128), lambda i: (0, i)),
        out_shape=jax.ShapeDtypeStruct((NNB, NPTS), jnp.int32),
        scratch_shapes=[pltpu.VMEM((NPTS, 128), F32)],
    )(p16b, p16bT, p16, p16T)


CW = 384


def _make_gather():
    info = plsc.get_sparse_core_info()
    ncores, nsub = info.num_cores, info.num_subcores
    nworkers = ncores * nsub
    rows_per = NROWS // nworkers
    chunk = 128
    nchunks = rows_per // chunk
    mesh = plsc.VectorSubcoreMesh(core_axis_name="c", subcore_axis_name="s")

    @functools.partial(
        pl.kernel, mesh=mesh,
        out_type=jax.ShapeDtypeStruct((NROWS, CW), F32),
        scratch_types=[
            pltpu.VMEM((chunk,), jnp.int32),
            pltpu.VMEM((chunk, CW), F32),
            pltpu.SemaphoreType.DMA,
        ],
    )
    def gather(xp_hbm, idx_hbm, gxp_hbm, idx_v, rows_v, sem1):
        wid = lax.axis_index("s") * ncores + lax.axis_index("c")

        def body(c, carry):
            base = wid * rows_per + c * chunk
            pltpu.sync_copy(idx_hbm.at[pl.ds(base, chunk)], idx_v)
            pltpu.async_copy(xp_hbm.at[idx_v], rows_v, sem1).wait()
            pltpu.sync_copy(rows_v, gxp_hbm.at[pl.ds(base, chunk)])
            return carry

        lax.fori_loop(0, nchunks, body, 0)

    return gather


def _pr_block(gp_ref, pq_ref, w1t_ref, a1_ref, c1_ref, w2t_ref, b2_ref):
    gp3 = gp_ref[...][:, :16].reshape(BP, NNB, 16) - pq_ref[...][:, None, :]
    y1 = _mm(gp3.reshape(BR, 16), w1t_ref[...])
    z1 = jnp.maximum(y1 * a1_ref[...] + c1_ref[...], 0.0)
    return _mmb(z1, w2t_ref[...]) + b2_ref[...]


def _acc_stats(s_ref, vals, width):
    s = jnp.sum(vals, axis=0, keepdims=True)
    q = jnp.sum(vals * vals, axis=0, keepdims=True)
    part = jnp.concatenate([s, q, jnp.zeros((6, width), F32)], axis=0)

    @pl.when(pl.program_id(0) == 0)
    def _():
        s_ref[...] = jnp.zeros_like(s_ref)

    s_ref[...] += part


def _stats1_body(gp_ref, pq_ref, w1t_ref, s_ref):
    gp3 = gp_ref[...][:, :16].reshape(BP, NNB, 16) - pq_ref[...][:, None, :]
    y1 = _mm(gp3.reshape(BR, 16), w1t_ref[...])
    _acc_stats(s_ref, y1, 16)


def _run_stats1(g_p, p16, W1T):
    return pl.pallas_call(
        _stats1_body,
        grid=(NPTS // BP,),
        in_specs=[
            pl.BlockSpec((BR, 128), lambda i: (i, 2)),
            pl.BlockSpec((BP, 16), lambda i: (i, 0)),
            pl.BlockSpec((16, 16), lambda i: (0, 0)),
        ],
        out_specs=pl.BlockSpec((8, 16), lambda i: (0, 0)),
        out_shape=jax.ShapeDtypeStruct((8, 16), F32),
    )(g_p, p16, W1T)


def _w0_block(gx_ref, gp_ref, xq_ref, pq_ref, wkt_ref, bk_ref,
              w1t_ref, a1_ref, c1_ref, w2t_ref, b2_ref):
    g_k = _mmb(gx_ref[...], wkt_ref[...]) + bk_ref[...]
    p_r = _pr_block(gp_ref, pq_ref, w1t_ref, a1_ref, c1_ref, w2t_ref, b2_ref)
    w03 = (g_k.reshape(BP, NNB, CF) - xq_ref[...][:, None, :]
           + p_r.reshape(BP, NNB, CF))
    return w03.reshape(BR, CF), p_r


def _alpha_body(gx_ref, gp_ref, xq_ref, pq_ref, wkt_ref, bk_ref,
                w1t_ref, a1_ref, c1_ref, w2t_ref, b2_ref, s_ref):
    w0, _ = _w0_block(gx_ref, gp_ref, xq_ref, pq_ref, wkt_ref, bk_ref,
                      w1t_ref, a1_ref, c1_ref, w2t_ref, b2_ref)
    _acc_stats(s_ref, w0, CF)


def _big_specs():
    return [
        pl.BlockSpec((BR, CF), lambda i: (i, 0)),
        pl.BlockSpec((BR, 128), lambda i: (i, 2)),
        pl.BlockSpec((BP, CF), lambda i: (i, 0)),
        pl.BlockSpec((BP, 16), lambda i: (i, 0)),
        pl.BlockSpec((CF, CF), lambda i: (0, 0)),
        pl.BlockSpec((1, CF), lambda i: (0, 0)),
        pl.BlockSpec((16, 16), lambda i: (0, 0)),
        pl.BlockSpec((1, 16), lambda i: (0, 0)),
        pl.BlockSpec((1, 16), lambda i: (0, 0)),
        pl.BlockSpec((16, CF), lambda i: (0, 0)),
        pl.BlockSpec((1, CF), lambda i: (0, 0)),
    ]


def _run_alpha(args):
    return pl.pallas_call(
        _alpha_body,
        grid=(NPTS // BP,),
        in_specs=_big_specs(),
        out_specs=pl.BlockSpec((8, CF), lambda i: (0, 0)),
        out_shape=jax.ShapeDtypeStruct((8, CF), F32),
    )(*args)


def _beta_body(gx_ref, gp_ref, xq_ref, pq_ref, wkt_ref, bk_ref,
               w1t_ref, a1_ref, c1_ref, w2t_ref, b2_ref,
               a2_ref, c2_ref, wc1t_ref, bc1_ref, w1o_ref, s_ref):
    w0, _ = _w0_block(gx_ref, gp_ref, xq_ref, pq_ref, wkt_ref, bk_ref,
                      w1t_ref, a1_ref, c1_ref, w2t_ref, b2_ref)
    z2 = jnp.maximum(w0 * a2_ref[...] + c2_ref[...], 0.0)
    w1 = _mmb(z2, wc1t_ref[...]) + bc1_ref[...]
    w1o_ref[...] = w1
    _acc_stats(s_ref, w1, 32)


def _run_beta(args):
    specs = _big_specs() + [
        pl.BlockSpec((1, CF), lambda i: (0, 0)),
        pl.BlockSpec((1, CF), lambda i: (0, 0)),
        pl.BlockSpec((CF, 32), lambda i: (0, 0)),
        pl.BlockSpec((1, 32), lambda i: (0, 0)),
    ]
    return pl.pallas_call(
        _beta_body,
        grid=(NPTS // BP,),
        in_specs=specs,
        out_specs=[
            pl.BlockSpec((BR, 32), lambda i: (i, 0)),
            pl.BlockSpec((8, 32), lambda i: (0, 0)),
        ],
        out_shape=[
            jax.ShapeDtypeStruct((NROWS, 32), F32),
            jax.ShapeDtypeStruct((8, 32), F32),
        ],
    )(*args)


def _gamma_body(w1_ref, gx_ref, gp_ref, pq_ref, wvt_ref, bv_ref,
                w1t_ref, a1_ref, c1_ref, w2t_ref, b2_ref,
                a3_ref, c3_ref, wc2t_ref, bc2_ref, o_ref):
    z3 = jnp.maximum(w1_ref[...] * a3_ref[...] + c3_ref[...], 0.0)
    w2 = _mm(z3, wc2t_ref[...]) + bc2_ref[...]
    w23 = w2.reshape(BP, NNB, 32)
    mx = jnp.max(w23, axis=1, keepdims=True)
    ex = jnp.exp(w23 - mx)
    att = ex / jnp.sum(ex, axis=1, keepdims=True)
    att_t = jnp.concatenate([att] * 8, axis=2)
    g_v = _mmb(gx_ref[...], wvt_ref[...]) + bv_ref[...]
    p_r = _pr_block(gp_ref, pq_ref, w1t_ref, a1_ref, c1_ref, w2t_ref, b2_ref)
    h = (g_v + p_r).reshape(BP, NNB, CF)
    o_ref[...] = jnp.sum(h * att_t, axis=1)


def _run_gamma(w1, gx, gp, p16, WvT, bv_row, W1T, A1, C1, W2T, bp2_row,
               A3, C3, Wc2T, bc2_row):
    specs = [
        pl.BlockSpec((BR, 32), lambda i: (i, 0)),
        pl.BlockSpec((BR, CF), lambda i: (i, 0)),
        pl.BlockSpec((BR, 128), lambda i: (i, 2)),
        pl.BlockSpec((BP, 16), lambda i: (i, 0)),
        pl.BlockSpec((CF, CF), lambda i: (0, 0)),
        pl.BlockSpec((1, CF), lambda i: (0, 0)),
        pl.BlockSpec((16, 16), lambda i: (0, 0)),
        pl.BlockSpec((1, 16), lambda i: (0, 0)),
        pl.BlockSpec((1, 16), lambda i: (0, 0)),
        pl.BlockSpec((16, CF), lambda i: (0, 0)),
        pl.BlockSpec((1, CF), lambda i: (0, 0)),
        pl.BlockSpec((1, 32), lambda i: (0, 0)),
        pl.BlockSpec((1, 32), lambda i: (0, 0)),
        pl.BlockSpec((32, 32), lambda i: (0, 0)),
        pl.BlockSpec((1, 32), lambda i: (0, 0)),
    ]
    return pl.pallas_call(
        _gamma_body,
        grid=(NPTS // BP,),
        in_specs=specs,
        out_specs=pl.BlockSpec((BP, CF), lambda i: (i, 0)),
        out_shape=jax.ShapeDtypeStruct((NPTS, CF), F32),
    )(w1, gx, gp, p16, WvT, bv_row, W1T, A1, C1, W2T, bp2_row,
      A3, C3, Wc2T, bc2_row)


def kernel(p, x, o, Wq, bq, Wk, bk, Wv, bv, Wp1, bp1, g1, be1, Wp2, bp2,
           g2, be2, Wc1, bc1, g3, be3, Wc2, bc2):
    row = lambda v: v[None, :]
    p16 = jnp.pad(p, ((0, 0), (0, 13)))
    p16b = p16.astype(jnp.bfloat16)
    W1T = jnp.pad(Wp1, ((0, 13), (0, 13))).T
    W2T = jnp.pad(Wp2, ((0, 0), (0, 13))).T

    x_q = _run_xq(x, Wq.T, row(bq))
    idxT = _run_knn(p16b, p16b.T, p16, p16.T)
    idxf = idxT.T.reshape(-1)

    xp = jnp.concatenate([x, p16, jnp.zeros((NPTS, CW - CF - 16), F32)],
                         axis=1)
    g_xp = _make_gather()(xp, idxf)

    M = float(NROWS)
    st1 = _run_stats1(g_xp, p16, W1T)
    m1 = st1[0] / M
    v1 = st1[1] / M - m1 * m1
    A1 = jnp.pad(g1, (0, 13)) / jnp.sqrt(v1 + 1e-5)
    C1 = jnp.pad(be1, (0, 13)) - m1 * A1

    big = (g_xp, g_xp, x_q, p16, Wk.T, row(bk), W1T, row(A1), row(C1),
           W2T, row(bp2))
    st2 = _run_alpha(big)
    m2 = st2[0] / M
    v2 = st2[1] / M - m2 * m2
    A2 = g2 / jnp.sqrt(v2 + 1e-5)
    C2 = be2 - m2 * A2

    w1, st3 = _run_beta(big + (row(A2), row(C2), Wc1.T, row(bc1)))
    m3 = st3[0, :32] / M
    v3 = st3[1, :32] / M - m3 * m3
    A3 = g3 / jnp.sqrt(v3 + 1e-5)
    C3 = be3 - m3 * A3

    out = _run_gamma(w1, g_xp, g_xp, p16, Wv.T, row(bv), W1T, row(A1), row(C1),
                     W2T, row(bp2), row(A3), row(C3), Wc2.T, row(bc2))
    return out


# speedup vs baseline: 2.3727x; 1.0568x over previous
import functools

import jax
import jax.numpy as jnp
from jax import lax
from jax.experimental import pallas as pl
from jax.experimental.pallas import tpu as pltpu
from jax.experimental.pallas import tpu_sc as plsc

NPTS = 8192
NNB = 16
CF = 256
NROWS = NPTS * NNB
BP = 128
BR = BP * NNB
HI = lax.Precision.HIGHEST
F32 = jnp.float32
BIGF = 3.0e38
BIGI = 2**30


def _mm(a, b):
    return jax.lax.dot_general(a, b, (((1,), (0,)), ((), ())),
                               preferred_element_type=F32, precision=HI)


def _mmb(a, b):
    return jax.lax.dot_general(a.astype(jnp.bfloat16), b.astype(jnp.bfloat16),
                               (((1,), (0,)), ((), ())),
                               preferred_element_type=F32)


def _xq_body(x_ref, wt_ref, b_ref, o_ref):
    o_ref[...] = _mmb(x_ref[...], wt_ref[...]) + b_ref[...]


def _run_xq(x, WqT, bq_row):
    return pl.pallas_call(
        _xq_body,
        grid=(16,),
        in_specs=[
            pl.BlockSpec((512, CF), lambda i: (i, 0)),
            pl.BlockSpec((CF, CF), lambda i: (0, 0)),
            pl.BlockSpec((1, CF), lambda i: (0, 0)),
        ],
        out_specs=pl.BlockSpec((512, CF), lambda i: (i, 0)),
        out_shape=jax.ShapeDtypeStruct((NPTS, CF), F32),
    )(x, WqT, bq_row)


def _knn_body(pb_ref, pbT_ref, pf_ref, pfT_ref, o_ref, d2_ref):
    dot = jax.lax.dot_general(pb_ref[...], pbT_ref[...],
                              (((1,), (0,)), ((), ())),
                              preferred_element_type=F32)
    pf = pf_ref[...]
    px, py, pz = pf[:, 0:1], pf[:, 1:2], pf[:, 2:3]
    sqc = (px * px + py * py) + pz * pz
    pfT = pfT_ref[...]
    qx, qy, qz = pfT[0:1, :], pfT[1:2, :], pfT[2:3, :]
    sqr = (qx * qx + qy * qy) + qz * qz
    d2_ref[...] = (sqc + sqr) - 2.0 * dot
    rows = jax.lax.broadcasted_iota(jnp.int32, (NPTS, 128), 0)

    def rnd(k, im_prev):
        d2m = jnp.where(rows == im_prev, BIGF, d2_ref[...])
        d2_ref[...] = d2m
        im = jnp.argmin(d2m, axis=0).reshape(1, 128)
        o_ref[pl.ds(k, 1), :] = im
        return im

    lax.fori_loop(0, NNB, rnd, jnp.full((1, 128), -1, jnp.int32))


def _run_knn(p16b, p16bT, p16, p16T):
    return pl.pallas_call(
        _knn_body,
        grid=(64,),
        in_specs=[
            pl.BlockSpec((NPTS, 16), lambda i: (0, 0)),
            pl.BlockSpec((16, 128), lambda i: (0, i)),
            pl.BlockSpec((NPTS, 16), lambda i: (0, 0)),
            pl.BlockSpec((16, 128), lambda i: (0, i)),
        ],
        out_specs=pl.BlockSpec((16, 128), lambda i: (0, i)),
        out_shape=jax.ShapeDtypeStruct((NNB, NPTS), jnp.int32),
        scratch_shapes=[pltpu.VMEM((NPTS, 128), F32)],
    )(p16b, p16bT, p16, p16T)


CW = 384


def _make_gather():
    info = plsc.get_sparse_core_info()
    ncores, nsub = info.num_cores, info.num_subcores
    nworkers = ncores * nsub
    rows_per = NROWS // nworkers
    chunk = 128
    nchunks = rows_per // chunk
    mesh = plsc.VectorSubcoreMesh(core_axis_name="c", subcore_axis_name="s")

    @functools.partial(
        pl.kernel, mesh=mesh,
        out_type=jax.ShapeDtypeStruct((NROWS, CW), F32),
        scratch_types=[
            pltpu.VMEM((chunk,), jnp.int32),
            pltpu.VMEM((chunk, CW), F32),
            pltpu.SemaphoreType.DMA,
        ],
    )
    def gather(xp_hbm, idx_hbm, gxp_hbm, idx_v, rows_v, sem1):
        wid = lax.axis_index("s") * ncores + lax.axis_index("c")

        def body(c, carry):
            base = wid * rows_per + c * chunk
            pltpu.sync_copy(idx_hbm.at[pl.ds(base, chunk)], idx_v)
            pltpu.async_copy(xp_hbm.at[idx_v], rows_v, sem1).wait()
            pltpu.sync_copy(rows_v, gxp_hbm.at[pl.ds(base, chunk)])
            return carry

        lax.fori_loop(0, nchunks, body, 0)

    return gather


def _pr_block(y1_ref, a1_ref, c1_ref, w2t_ref, b2_ref):
    z1 = jnp.maximum(y1_ref[...] * a1_ref[...] + c1_ref[...], 0.0)
    return _mmb(z1, w2t_ref[...]) + b2_ref[...]


def _acc_stats(s_ref, vals, width):
    s = jnp.sum(vals, axis=0, keepdims=True)
    q = jnp.sum(vals * vals, axis=0, keepdims=True)
    part = jnp.concatenate([s, q, jnp.zeros((6, width), F32)], axis=0)

    @pl.when(pl.program_id(0) == 0)
    def _():
        s_ref[...] = jnp.zeros_like(s_ref)

    s_ref[...] += part


def _stats1_body(gp_ref, pq_ref, w1t_ref, y1o_ref, s_ref):
    gp3 = gp_ref[...][:, :16].reshape(BP, NNB, 16) - pq_ref[...][:, None, :]
    y1 = _mm(gp3.reshape(BR, 16), w1t_ref[...])
    y1o_ref[...] = y1
    _acc_stats(s_ref, y1, 16)


def _run_stats1(g_p, p16, W1T):
    return pl.pallas_call(
        _stats1_body,
        grid=(NPTS // BP,),
        in_specs=[
            pl.BlockSpec((BR, 128), lambda i: (i, 2)),
            pl.BlockSpec((BP, 16), lambda i: (i, 0)),
            pl.BlockSpec((16, 16), lambda i: (0, 0)),
        ],
        out_specs=[
            pl.BlockSpec((BR, 16), lambda i: (i, 0)),
            pl.BlockSpec((8, 16), lambda i: (0, 0)),
        ],
        out_shape=[
            jax.ShapeDtypeStruct((NROWS, 16), F32),
            jax.ShapeDtypeStruct((8, 16), F32),
        ],
    )(g_p, p16, W1T)


def _w0_block(gx_ref, y1_ref, xq_ref, wkt_ref, bk_ref,
              a1_ref, c1_ref, w2t_ref, b2_ref):
    g_k = _mmb(gx_ref[...], wkt_ref[...]) + bk_ref[...]
    p_r = _pr_block(y1_ref, a1_ref, c1_ref, w2t_ref, b2_ref)
    w03 = (g_k.reshape(BP, NNB, CF) - xq_ref[...][:, None, :]
           + p_r.reshape(BP, NNB, CF))
    return w03.reshape(BR, CF), p_r


def _alpha_body(gx_ref, y1_ref, xq_ref, wkt_ref, bk_ref,
                a1_ref, c1_ref, w2t_ref, b2_ref, s_ref):
    w0, _ = _w0_block(gx_ref, y1_ref, xq_ref, wkt_ref, bk_ref,
                      a1_ref, c1_ref, w2t_ref, b2_ref)
    _acc_stats(s_ref, w0, CF)


def _big_specs():
    return [
        pl.BlockSpec((BR, CF), lambda i: (i, 0)),
        pl.BlockSpec((BR, 16), lambda i: (i, 0)),
        pl.BlockSpec((BP, CF), lambda i: (i, 0)),
        pl.BlockSpec((CF, CF), lambda i: (0, 0)),
        pl.BlockSpec((1, CF), lambda i: (0, 0)),
        pl.BlockSpec((1, 16), lambda i: (0, 0)),
        pl.BlockSpec((1, 16), lambda i: (0, 0)),
        pl.BlockSpec((16, CF), lambda i: (0, 0)),
        pl.BlockSpec((1, CF), lambda i: (0, 0)),
    ]


def _run_alpha(args):
    return pl.pallas_call(
        _alpha_body,
        grid=(NPTS // BP,),
        in_specs=_big_specs(),
        out_specs=pl.BlockSpec((8, CF), lambda i: (0, 0)),
        out_shape=jax.ShapeDtypeStruct((8, CF), F32),
    )(*args)


def _beta_body(gx_ref, y1_ref, xq_ref, wkt_ref, bk_ref,
               a1_ref, c1_ref, w2t_ref, b2_ref,
               a2_ref, c2_ref, wc1t_ref, bc1_ref, w1o_ref, s_ref):
    w0, _ = _w0_block(gx_ref, y1_ref, xq_ref, wkt_ref, bk_ref,
                      a1_ref, c1_ref, w2t_ref, b2_ref)
    z2 = jnp.maximum(w0 * a2_ref[...] + c2_ref[...], 0.0)
    w1 = _mmb(z2, wc1t_ref[...]) + bc1_ref[...]
    w1o_ref[...] = w1
    _acc_stats(s_ref, w1, 32)


def _run_beta(args):
    specs = _big_specs() + [
        pl.BlockSpec((1, CF), lambda i: (0, 0)),
        pl.BlockSpec((1, CF), lambda i: (0, 0)),
        pl.BlockSpec((CF, 32), lambda i: (0, 0)),
        pl.BlockSpec((1, 32), lambda i: (0, 0)),
    ]
    return pl.pallas_call(
        _beta_body,
        grid=(NPTS // BP,),
        in_specs=specs,
        out_specs=[
            pl.BlockSpec((BR, 32), lambda i: (i, 0)),
            pl.BlockSpec((8, 32), lambda i: (0, 0)),
        ],
        out_shape=[
            jax.ShapeDtypeStruct((NROWS, 32), F32),
            jax.ShapeDtypeStruct((8, 32), F32),
        ],
    )(*args)


def _gamma_body(w1_ref, gx_ref, y1_ref, wvt_ref, bv_ref,
                a1_ref, c1_ref, w2t_ref, b2_ref,
                a3_ref, c3_ref, wc2t_ref, bc2_ref, o_ref):
    z3 = jnp.maximum(w1_ref[...] * a3_ref[...] + c3_ref[...], 0.0)
    w2 = _mm(z3, wc2t_ref[...]) + bc2_ref[...]
    w23 = w2.reshape(BP, NNB, 32)
    mx = jnp.max(w23, axis=1, keepdims=True)
    ex = jnp.exp(w23 - mx)
    att = ex / jnp.sum(ex, axis=1, keepdims=True)
    att_t = jnp.concatenate([att] * 8, axis=2)
    g_v = _mmb(gx_ref[...], wvt_ref[...]) + bv_ref[...]
    p_r = _pr_block(y1_ref, a1_ref, c1_ref, w2t_ref, b2_ref)
    h = (g_v + p_r).reshape(BP, NNB, CF)
    o_ref[...] = jnp.sum(h * att_t, axis=1)


def _run_gamma(w1, gx, y1, WvT, bv_row, A1, C1, W2T, bp2_row,
               A3, C3, Wc2T, bc2_row):
    specs = [
        pl.BlockSpec((BR, 32), lambda i: (i, 0)),
        pl.BlockSpec((BR, CF), lambda i: (i, 0)),
        pl.BlockSpec((BR, 16), lambda i: (i, 0)),
        pl.BlockSpec((CF, CF), lambda i: (0, 0)),
        pl.BlockSpec((1, CF), lambda i: (0, 0)),
        pl.BlockSpec((1, 16), lambda i: (0, 0)),
        pl.BlockSpec((1, 16), lambda i: (0, 0)),
        pl.BlockSpec((16, CF), lambda i: (0, 0)),
        pl.BlockSpec((1, CF), lambda i: (0, 0)),
        pl.BlockSpec((1, 32), lambda i: (0, 0)),
        pl.BlockSpec((1, 32), lambda i: (0, 0)),
        pl.BlockSpec((32, 32), lambda i: (0, 0)),
        pl.BlockSpec((1, 32), lambda i: (0, 0)),
    ]
    return pl.pallas_call(
        _gamma_body,
        grid=(NPTS // BP,),
        in_specs=specs,
        out_specs=pl.BlockSpec((BP, CF), lambda i: (i, 0)),
        out_shape=jax.ShapeDtypeStruct((NPTS, CF), F32),
    )(w1, gx, y1, WvT, bv_row, A1, C1, W2T, bp2_row,
      A3, C3, Wc2T, bc2_row)


def kernel(p, x, o, Wq, bq, Wk, bk, Wv, bv, Wp1, bp1, g1, be1, Wp2, bp2,
           g2, be2, Wc1, bc1, g3, be3, Wc2, bc2):
    row = lambda v: v[None, :]
    p16 = jnp.pad(p, ((0, 0), (0, 13)))
    p16b = p16.astype(jnp.bfloat16)
    W1T = jnp.pad(Wp1, ((0, 13), (0, 13))).T
    W2T = jnp.pad(Wp2, ((0, 0), (0, 13))).T

    x_q = _run_xq(x, Wq.T, row(bq))
    idxT = _run_knn(p16b, p16b.T, p16, p16.T)
    idxf = idxT.T.reshape(-1)

    xp = jnp.concatenate([x, p16, jnp.zeros((NPTS, CW - CF - 16), F32)],
                         axis=1)
    g_xp = _make_gather()(xp, idxf)

    M = float(NROWS)
    y1, st1 = _run_stats1(g_xp, p16, W1T)
    m1 = st1[0] / M
    v1 = st1[1] / M - m1 * m1
    A1 = jnp.pad(g1, (0, 13)) / jnp.sqrt(v1 + 1e-5)
    C1 = jnp.pad(be1, (0, 13)) - m1 * A1

    big = (g_xp, y1, x_q, Wk.T, row(bk), row(A1), row(C1),
           W2T, row(bp2))
    st2 = _run_alpha(big)
    m2 = st2[0] / M
    v2 = st2[1] / M - m2 * m2
    A2 = g2 / jnp.sqrt(v2 + 1e-5)
    C2 = be2 - m2 * A2

    w1, st3 = _run_beta(big + (row(A2), row(C2), Wc1.T, row(bc1)))
    m3 = st3[0, :32] / M
    v3 = st3[1, :32] / M - m3 * m3
    A3 = g3 / jnp.sqrt(v3 + 1e-5)
    C3 = be3 - m3 * A3

    out = _run_gamma(w1, g_xp, y1, Wv.T, row(bv), row(A1), row(C1),
                     W2T, row(bp2), row(A3), row(C3), Wc2.T, row(bc2))
    return out
